# Initial kernel scaffold; baseline (speedup 1.0000x reference)
import jax
import jax.numpy as jnp
from jax.experimental import pallas as pl


def kernel(nodes, edge_index, table):
    raise NotImplementedError("write your pallas kernel here")


# speedup vs baseline: 10.5645x; 10.5645x over previous
import jax
import jax.numpy as jnp
from jax import lax
from jax.experimental import pallas as pl
from jax.experimental.pallas import tpu as pltpu
from jax.experimental.pallas import tpu_sc as plsc

NC = 2
NS = 16
B = 10000
D = 128
DH = D // NC
E = 320000
BURST = 128
CH = 12
NCHUNK = 14
NPAIR = NCHUNK // 2
NBURST = NCHUNK * CH
WPW = NBURST * BURST
TOT = NS * WPW
PAD_DST = B
ACC_ROWS = 10240
RPW = ACC_ROWS // NS
TPW = B // NS
CW = 16


def _sc_body(tableT_h, srcI_h, dstI_h, ones_h, zrow_h, zcnt_h,
             psum_h, pcnt_h,
             srcC0, srcC1, dstC0, dstC1, ones_v, rows0, rows1,
             table_s, acc_s, cnt_s,
             g0, g1, s0, s1, c0, c1, i0, i1):
    c = lax.axis_index("c")
    s = lax.axis_index("s")

    SRC = (srcC0, srcC1)
    DST = (dstC0, dstC1)
    ROWS = (rows0, rows1)
    G = (g0, g1)
    S = (s0, s1)
    CS = (c0, c1)
    I = (i0, i1)

    pltpu.sync_copy(tableT_h.at[c].at[pl.ds(s * TPW, TPW)],
                    table_s.at[pl.ds(s * TPW, TPW)])
    pltpu.sync_copy(zrow_h, acc_s.at[pl.ds(s * RPW, RPW)])
    pltpu.sync_copy(zcnt_h, cnt_s.at[pl.ds(s * RPW, RPW)])
    pltpu.sync_copy(ones_h, ones_v)
    pltpu.sync_copy(srcI_h.at[s].at[pl.ds(0, CH)], srcC0)
    pltpu.sync_copy(dstI_h.at[s].at[pl.ds(0, CH)], dstC0)
    plsc.subcore_barrier()

    def g_start(cp, j, bp):
        pltpu.async_copy(table_s.at[SRC[cp].at[j]], ROWS[bp], G[bp])

    def g_wait(cp, j, bp):
        pltpu.make_async_copy(table_s.at[SRC[cp].at[j]], ROWS[bp], G[bp]).wait()

    def s_start(cp, j, bp):
        pltpu.async_copy(ROWS[bp], acc_s.at[DST[cp].at[j]], S[bp], add=True)
        pltpu.async_copy(ones_v, cnt_s.at[DST[cp].at[j]], CS[bp], add=True)

    def s_wait(cp, j, bp):
        pltpu.make_async_copy(ROWS[bp], acc_s.at[DST[cp].at[j]], S[bp]).wait()
        pltpu.make_async_copy(ones_v, cnt_s.at[DST[cp].at[j]], CS[bp]).wait()

    def pf_start(ci, buf):
        pltpu.async_copy(srcI_h.at[s].at[pl.ds(ci * CH, CH)], SRC[buf], I[buf])
        pltpu.async_copy(dstI_h.at[s].at[pl.ds(ci * CH, CH)], DST[buf], I[buf])

    def pf_wait(ci, buf):
        pltpu.make_async_copy(
            srcI_h.at[s].at[pl.ds(ci * CH, CH)], SRC[buf], I[buf]).wait()
        pltpu.make_async_copy(
            dstI_h.at[s].at[pl.ds(ci * CH, CH)], DST[buf], I[buf]).wait()

    def do_chunk(ci, p, first_chunk=False):
        for j in range(CH):
            bp = j % 2
            if first_chunk and j == 0:
                g_start(p, 0, 0)
            else:
                if j > 0:
                    s_wait(p, j - 1, 1 - bp)
                else:
                    s_wait(1 - p, CH - 1, 1 - bp)
            if j == 1:
                pf_start(ci + 1, 1 - p)
            if j == CH - 1:
                pf_wait(ci + 1, 1 - p)
                g_start(1 - p, 0, 1 - bp)
            else:
                g_start(p, j + 1, 1 - bp)
            g_wait(p, j, bp)
            s_start(p, j, bp)

    do_chunk(0, 0, first_chunk=True)
    do_chunk(1, 1)

    @pl.loop(1, NPAIR)
    def _(k):
        do_chunk(2 * k, 0)
        do_chunk(2 * k + 1, 1)

    g_wait(0, 0, 0)
    s_wait(1, CH - 1, 1)

    plsc.subcore_barrier()
    pltpu.sync_copy(acc_s.at[pl.ds(s * RPW, RPW)],
                    psum_h.at[c].at[pl.ds(s * RPW, RPW)])
    pltpu.sync_copy(cnt_s.at[pl.ds(s * RPW, RPW)],
                    pcnt_h.at[c].at[pl.ds(s * RPW, RPW)])


def _sc_aggregate(tableT, src_idx, dst_idx, ones, zrow, zcnt):
    mesh = plsc.VectorSubcoreMesh(core_axis_name="c", subcore_axis_name="s")
    return pl.kernel(
        _sc_body,
        compiler_params=pltpu.CompilerParams(use_tc_tiling_on_sc=False),
        out_type=[
            jax.ShapeDtypeStruct((NC, ACC_ROWS, DH), jnp.float32),
            jax.ShapeDtypeStruct((NC, ACC_ROWS, CW), jnp.float32),
        ],
        mesh=mesh,
        scratch_types=[
            pltpu.VMEM((CH, BURST), jnp.int32),
            pltpu.VMEM((CH, BURST), jnp.int32),
            pltpu.VMEM((CH, BURST), jnp.int32),
            pltpu.VMEM((CH, BURST), jnp.int32),
            pltpu.VMEM((BURST, CW), jnp.float32),
            pltpu.VMEM((BURST, DH), jnp.float32),
            pltpu.VMEM((BURST, DH), jnp.float32),
            pltpu.VMEM_SHARED((B, DH), jnp.float32),
            pltpu.VMEM_SHARED((ACC_ROWS, DH), jnp.float32),
            pltpu.VMEM_SHARED((ACC_ROWS, CW), jnp.float32),
            pltpu.SemaphoreType.DMA,
            pltpu.SemaphoreType.DMA,
            pltpu.SemaphoreType.DMA,
            pltpu.SemaphoreType.DMA,
            pltpu.SemaphoreType.DMA,
            pltpu.SemaphoreType.DMA,
            pltpu.SemaphoreType.DMA,
            pltpu.SemaphoreType.DMA,
        ],
    )(tableT, src_idx, dst_idx, ones, zrow, zcnt)


BLK = 400


def _combine_body(ps_ref, pc_ref, o_ref):
    cnt = jnp.maximum(pc_ref[0, :, 0], 1.0)
    inv = (1.0 / cnt)[:, None]
    o_ref[...] = jnp.concatenate(
        [ps_ref[0] * inv, ps_ref[1] * inv], axis=1)


def _combine(psum, pcnt):
    return pl.pallas_call(
        _combine_body,
        grid=(B // BLK,),
        in_specs=[
            pl.BlockSpec((NC, BLK, DH), lambda i: (0, i, 0)),
            pl.BlockSpec((1, BLK, CW), lambda i: (0, i, 0)),
        ],
        out_specs=pl.BlockSpec((BLK, D), lambda i: (i, 0)),
        out_shape=jax.ShapeDtypeStruct((B, D), jnp.float32),
    )(psum, pcnt)


def kernel(nodes, edge_index, table):
    dst = edge_index[0]
    src = edge_index[1]
    tableT = table.reshape(B, NC, DH).transpose(1, 0, 2)
    npad = TOT - (E + B)
    dst_all = jnp.concatenate(
        [dst, jnp.arange(B, dtype=jnp.int32),
         jnp.full((npad,), PAD_DST, jnp.int32)])
    src_all = jnp.concatenate(
        [src, nodes.astype(jnp.int32), jnp.zeros((npad,), jnp.int32)])
    src_idx = jnp.concatenate(
        [src_all.reshape(NS, NBURST, BURST),
         jnp.zeros((NS, CH, BURST), jnp.int32)], axis=1)
    dst_idx = jnp.concatenate(
        [dst_all.reshape(NS, NBURST, BURST),
         jnp.full((NS, CH, BURST), PAD_DST, jnp.int32)], axis=1)
    ones = jnp.ones((BURST, CW), jnp.float32)
    zrow = jnp.zeros((RPW, DH), jnp.float32)
    zcnt = jnp.zeros((RPW, CW), jnp.float32)
    psum, pcnt = _sc_aggregate(tableT, src_idx, dst_idx, ones, zrow, zcnt)
    return _combine(psum, pcnt)


# speedup vs baseline: 12.3275x; 1.1669x over previous
import jax
import jax.numpy as jnp
from jax import lax
from jax.experimental import pallas as pl
from jax.experimental.pallas import tpu as pltpu
from jax.experimental.pallas import tpu_sc as plsc

NC = 2
NS = 16
L = 16
B = 10000
D = 128
DH = D // NC
E = 320000
BURST = 128
CH = 12
NCHUNK = 14
NPAIR = NCHUNK // 2
NBURST = NCHUNK * CH
WPW = NBURST * BURST
TOT = NS * WPW
PAD_DST = B
ACC_ROWS = 10240
RPW = ACC_ROWS // NS
TPW = B // NS
NPIECE = 5
PJ = TPW // NPIECE
CW = 16


def _sc_body(table_h, srcI_h, dstI_h, ones_h, zrow_h, zcnt_h,
             out_h,
             srcC0, srcC1, dstC0, dstC1, ones_v, rows0, rows1, res_v, cnt_v,
             table_s, acc_s, cnt_s,
             g0, g1, s0, s1, c0, c1, i0, i1):
    c = lax.axis_index("c")
    s = lax.axis_index("s")

    SRC = (srcC0, srcC1)
    DST = (dstC0, dstC1)
    ROWS = (rows0, rows1)
    G = (g0, g1)
    S = (s0, s1)
    CS = (c0, c1)
    I = (i0, i1)

    pltpu.sync_copy(table_h.at[pl.ds(s * TPW, TPW), pl.ds(c * DH, DH)],
                    table_s.at[pl.ds(s * TPW, TPW)])
    pltpu.sync_copy(zrow_h, acc_s.at[pl.ds(s * RPW, RPW)])
    pltpu.sync_copy(zcnt_h, cnt_s.at[pl.ds(s * RPW, RPW)])
    pltpu.sync_copy(ones_h, ones_v)
    pltpu.sync_copy(srcI_h.at[s].at[pl.ds(0, CH)], srcC0)
    pltpu.sync_copy(dstI_h.at[s].at[pl.ds(0, CH)], dstC0)
    plsc.subcore_barrier()

    def g_start(cp, j, bp):
        pltpu.async_copy(table_s.at[SRC[cp].at[j]], ROWS[bp], G[bp])

    def g_wait(cp, j, bp):
        pltpu.make_async_copy(table_s.at[SRC[cp].at[j]], ROWS[bp], G[bp]).wait()

    def s_start(cp, j, bp):
        pltpu.async_copy(ROWS[bp], acc_s.at[DST[cp].at[j]], S[bp], add=True)
        pltpu.async_copy(ones_v, cnt_s.at[DST[cp].at[j]], CS[bp], add=True)

    def s_wait(cp, j, bp):
        pltpu.make_async_copy(ROWS[bp], acc_s.at[DST[cp].at[j]], S[bp]).wait()
        pltpu.make_async_copy(ones_v, cnt_s.at[DST[cp].at[j]], CS[bp]).wait()

    def pf_start(off, buf):
        pltpu.async_copy(srcI_h.at[s].at[pl.ds(off, CH)], SRC[buf], I[buf])
        pltpu.async_copy(dstI_h.at[s].at[pl.ds(off, CH)], DST[buf], I[buf])

    def pf_wait(off, buf):
        pltpu.make_async_copy(
            srcI_h.at[s].at[pl.ds(off, CH)], SRC[buf], I[buf]).wait()
        pltpu.make_async_copy(
            dstI_h.at[s].at[pl.ds(off, CH)], DST[buf], I[buf]).wait()

    def chunk_off(ci):
        return jnp.where(ci < NCHUNK, ci * CH, 0)

    def do_chunk(ci, p, first_chunk=False):
        off_next = chunk_off(ci + 1)
        for j in range(CH):
            bp = j % 2
            if first_chunk and j == 0:
                g_start(p, 0, 0)
            else:
                if j > 0:
                    s_wait(p, j - 1, 1 - bp)
                else:
                    s_wait(1 - p, CH - 1, 1 - bp)
            if j == 1:
                pf_start(off_next, 1 - p)
            if j == CH - 1:
                pf_wait(off_next, 1 - p)
                g_start(1 - p, 0, 1 - bp)
            else:
                g_start(p, j + 1, 1 - bp)
            g_wait(p, j, bp)
            s_start(p, j, bp)

    do_chunk(0, 0, first_chunk=True)
    do_chunk(1, 1)

    @pl.loop(1, NPAIR)
    def _(k):
        do_chunk(2 * k, 0)
        do_chunk(2 * k + 1, 1)

    g_wait(0, 0, 0)
    s_wait(1, CH - 1, 1)

    plsc.subcore_barrier()
    @pl.loop(0, NPIECE)
    def _(t):
        base = s * TPW + t * PJ
        pltpu.sync_copy(acc_s.at[pl.ds(base, PJ)], res_v)
        pltpu.sync_copy(cnt_s.at[pl.ds(base, PJ)], cnt_v)

        @pl.loop(0, PJ)
        def _(r):
            cnt = cnt_v[r, pl.ds(0, L)]
            inv = 1.0 / jnp.maximum(cnt, 1.0)
            for q in range(DH // L):
                res_v[r, pl.ds(q * L, L)] = res_v[r, pl.ds(q * L, L)] * inv

        pltpu.sync_copy(res_v,
                        out_h.at[pl.ds(base, PJ), pl.ds(c * DH, DH)])


def _sc_aggregate(table, src_idx, dst_idx, ones, zrow, zcnt):
    mesh = plsc.VectorSubcoreMesh(core_axis_name="c", subcore_axis_name="s")
    return pl.kernel(
        _sc_body,
        compiler_params=pltpu.CompilerParams(use_tc_tiling_on_sc=False),
        out_type=jax.ShapeDtypeStruct((B, D), jnp.float32),
        mesh=mesh,
        scratch_types=[
            pltpu.VMEM((CH, BURST), jnp.int32),
            pltpu.VMEM((CH, BURST), jnp.int32),
            pltpu.VMEM((CH, BURST), jnp.int32),
            pltpu.VMEM((CH, BURST), jnp.int32),
            pltpu.VMEM((BURST, CW), jnp.float32),
            pltpu.VMEM((BURST, DH), jnp.float32),
            pltpu.VMEM((BURST, DH), jnp.float32),
            pltpu.VMEM((PJ, DH), jnp.float32),
            pltpu.VMEM((PJ, CW), jnp.float32),
            pltpu.VMEM_SHARED((B, DH), jnp.float32),
            pltpu.VMEM_SHARED((ACC_ROWS, DH), jnp.float32),
            pltpu.VMEM_SHARED((ACC_ROWS, CW), jnp.float32),
            pltpu.SemaphoreType.DMA,
            pltpu.SemaphoreType.DMA,
            pltpu.SemaphoreType.DMA,
            pltpu.SemaphoreType.DMA,
            pltpu.SemaphoreType.DMA,
            pltpu.SemaphoreType.DMA,
            pltpu.SemaphoreType.DMA,
            pltpu.SemaphoreType.DMA,
        ],
    )(table, src_idx, dst_idx, ones, zrow, zcnt)


def kernel(nodes, edge_index, table):
    dst = edge_index[0]
    src = edge_index[1]
    npad = TOT - (E + B)
    dst_idx = jnp.concatenate(
        [dst, jnp.arange(B, dtype=jnp.int32),
         jnp.full((npad,), PAD_DST, jnp.int32)]).reshape(NS, NBURST, BURST)
    src_idx = jnp.concatenate(
        [src, nodes.astype(jnp.int32),
         jnp.zeros((npad,), jnp.int32)]).reshape(NS, NBURST, BURST)
    ones = jnp.ones((BURST, CW), jnp.float32)
    zrow = jnp.zeros((RPW, DH), jnp.float32)
    zcnt = jnp.zeros((RPW, CW), jnp.float32)
    return _sc_aggregate(table, src_idx, dst_idx, ones, zrow, zcnt)


# speedup vs baseline: 14.4714x; 1.1739x over previous
import jax
import jax.numpy as jnp
from jax import lax
from jax.experimental import pallas as pl
from jax.experimental.pallas import tpu as pltpu
from jax.experimental.pallas import tpu_sc as plsc

NC = 2
NS = 16
L = 16
B = 10000
D = 128
DH = D // NC
E = 320000
BURST = 128
CH = 12
NCHUNK = 14
NPAIR = NCHUNK // 2
NBURST = NCHUNK * CH
WPW = NBURST * BURST
TOT = NS * WPW
PAD_DST = B
ACC_ROWS = 10240
RPW = ACC_ROWS // NS
TPW = B // NS
NPIECE = 5
PJ = TPW // NPIECE
CW = 16
CROWS = ACC_ROWS // CW
CZPW = CROWS // NS


def _sc_body(table_h, srcI_h, dstI_h, zeros_h,
             out_h,
             srcC0, srcC1, dstC0, dstC1, rows0, rows1, res_v, cnt_v,
             hist_v, idxv,
             table_s, acc_s, cnt_s,
             g0, g1, s0, s1, i0, i1):
    c = lax.axis_index("c")
    s = lax.axis_index("s")

    SRC = (srcC0, srcC1)
    DST = (dstC0, dstC1)
    ROWS = (rows0, rows1)
    G = (g0, g1)
    S = (s0, s1)
    I = (i0, i1)

    ones16 = jnp.ones((L,), jnp.float32)

    pltpu.sync_copy(table_h.at[pl.ds(s * TPW, TPW), pl.ds(c * DH, DH)],
                    table_s.at[pl.ds(s * TPW, TPW)])
    pltpu.sync_copy(zeros_h, acc_s.at[pl.ds(s * RPW, RPW)])
    pltpu.sync_copy(zeros_h.at[pl.ds(0, CZPW), pl.ds(0, CW)],
                    cnt_s.at[pl.ds(s * CZPW, CZPW)])
    pltpu.sync_copy(zeros_h.at[pl.ds(0, CROWS), pl.ds(0, CW)], hist_v)
    for t in range(CROWS // L):
        idxv[pl.ds(t * L, L)] = lax.iota(jnp.int32, L) + (t * L)
    pltpu.sync_copy(srcI_h.at[s].at[pl.ds(0, CH)], srcC0)
    pltpu.sync_copy(dstI_h.at[s].at[pl.ds(0, CH)], dstC0)
    plsc.subcore_barrier()

    def g_start(cp, j, bp):
        pltpu.async_copy(table_s.at[SRC[cp].at[j]], ROWS[bp], G[bp])

    def g_wait(cp, j, bp):
        pltpu.make_async_copy(table_s.at[SRC[cp].at[j]], ROWS[bp], G[bp]).wait()

    def s_start(cp, j, bp):
        pltpu.async_copy(ROWS[bp], acc_s.at[DST[cp].at[j]], S[bp], add=True)

    def s_wait(cp, j, bp):
        pltpu.make_async_copy(ROWS[bp], acc_s.at[DST[cp].at[j]], S[bp]).wait()

    def count(cp, j):
        for t in range(BURST // L):
            dv = DST[cp][j, pl.ds(t * L, L)]
            row = lax.shift_right_logical(dv, 4)
            lane = lax.bitwise_and(dv, 15)
            plsc.addupdate_scatter(hist_v, [row, lane], ones16)

    def pf_start(off, buf):
        pltpu.async_copy(srcI_h.at[s].at[pl.ds(off, CH)], SRC[buf], I[buf])
        pltpu.async_copy(dstI_h.at[s].at[pl.ds(off, CH)], DST[buf], I[buf])

    def pf_wait(off, buf):
        pltpu.make_async_copy(
            srcI_h.at[s].at[pl.ds(off, CH)], SRC[buf], I[buf]).wait()
        pltpu.make_async_copy(
            dstI_h.at[s].at[pl.ds(off, CH)], DST[buf], I[buf]).wait()

    def chunk_off(ci):
        return jnp.where(ci < NCHUNK, ci * CH, 0)

    def do_chunk(ci, p, first_chunk=False):
        off_next = chunk_off(ci + 1)
        for j in range(CH):
            bp = j % 2
            if first_chunk and j == 0:
                g_start(p, 0, 0)
            else:
                if j > 0:
                    s_wait(p, j - 1, 1 - bp)
                else:
                    s_wait(1 - p, CH - 1, 1 - bp)
            if j == 1:
                pf_start(off_next, 1 - p)
            if j == CH - 1:
                pf_wait(off_next, 1 - p)
                g_start(1 - p, 0, 1 - bp)
            else:
                g_start(p, j + 1, 1 - bp)
            g_wait(p, j, bp)
            s_start(p, j, bp)
            count(p, j)

    do_chunk(0, 0, first_chunk=True)
    do_chunk(1, 1)

    @pl.loop(1, NPAIR)
    def _(k):
        do_chunk(2 * k, 0)
        do_chunk(2 * k + 1, 1)

    g_wait(0, 0, 0)
    s_wait(1, CH - 1, 1)

    pltpu.sync_copy(hist_v, cnt_s.at[idxv], add=True)

    plsc.subcore_barrier()
    @pl.loop(0, NPIECE)
    def _(t):
        base = s * TPW + t * PJ
        crow0 = base // CW
        pltpu.sync_copy(acc_s.at[pl.ds(base, PJ)], res_v)
        pltpu.sync_copy(cnt_s.at[pl.ds(crow0, PJ // CW + 2)], cnt_v)

        @pl.loop(0, PJ)
        def _(r):
            flat = base + r
            rowp = flat // CW - crow0
            lanevec = jnp.broadcast_to(lax.rem(flat, CW), (L,))
            cnt = plsc.load_gather(cnt_v, [jnp.broadcast_to(rowp, (L,)),
                                           lanevec])
            inv = 1.0 / jnp.maximum(cnt, 1.0)
            for q in range(DH // L):
                res_v[r, pl.ds(q * L, L)] = res_v[r, pl.ds(q * L, L)] * inv

        pltpu.sync_copy(res_v,
                        out_h.at[pl.ds(base, PJ), pl.ds(c * DH, DH)])


def _sc_aggregate(table, src_idx, dst_idx, zeros):
    mesh = plsc.VectorSubcoreMesh(core_axis_name="c", subcore_axis_name="s")
    return pl.kernel(
        _sc_body,
        compiler_params=pltpu.CompilerParams(use_tc_tiling_on_sc=False,
                                             needs_layout_passes=False),
        out_type=jax.ShapeDtypeStruct((B, D), jnp.float32),
        mesh=mesh,
        scratch_types=[
            pltpu.VMEM((CH, BURST), jnp.int32),
            pltpu.VMEM((CH, BURST), jnp.int32),
            pltpu.VMEM((CH, BURST), jnp.int32),
            pltpu.VMEM((CH, BURST), jnp.int32),
            pltpu.VMEM((BURST, DH), jnp.float32),
            pltpu.VMEM((BURST, DH), jnp.float32),
            pltpu.VMEM((PJ, DH), jnp.float32),
            pltpu.VMEM((PJ // CW + 2, CW), jnp.float32),
            pltpu.VMEM((CROWS, CW), jnp.float32),
            pltpu.VMEM((CROWS,), jnp.int32),
            pltpu.VMEM_SHARED((B, DH), jnp.float32),
            pltpu.VMEM_SHARED((ACC_ROWS, DH), jnp.float32),
            pltpu.VMEM_SHARED((CROWS, CW), jnp.float32),
            pltpu.SemaphoreType.DMA,
            pltpu.SemaphoreType.DMA,
            pltpu.SemaphoreType.DMA,
            pltpu.SemaphoreType.DMA,
            pltpu.SemaphoreType.DMA,
            pltpu.SemaphoreType.DMA,
        ],
    )(table, src_idx, dst_idx, zeros)


def kernel(nodes, edge_index, table):
    dst = edge_index[0]
    src = edge_index[1]
    npad = TOT - (E + B)
    dst_idx = jnp.concatenate(
        [dst, jnp.arange(B, dtype=jnp.int32),
         jnp.full((npad,), PAD_DST, jnp.int32)]).reshape(NS, NBURST, BURST)
    src_idx = jnp.concatenate(
        [src, nodes.astype(jnp.int32),
         jnp.zeros((npad,), jnp.int32)]).reshape(NS, NBURST, BURST)
    zeros = jnp.zeros((RPW, DH), jnp.float32)
    return _sc_aggregate(table, src_idx, dst_idx, zeros)
